# baseline (speedup 1.0000x reference)
import jax
import jax.numpy as jnp
from jax.experimental import pallas as pl
from jax.experimental.pallas import tpu as pltpu

B, TOTAL, D_Z, D_PAIR, D_LM, M, H = 16, 32768, 128, 128, 768, 16, 128

CHUNK = 2048
N_P1 = TOTAL // CHUNK
NEG = -1e30

BLK = 768
N_P2 = (M * D_LM) // BLK


def _fused_kernel(sinfo_ref, st_ref, en_ref, zs_ref, wv_ref, wu_ref, w1_ref,
                  b1_ref, b2_ref, zn_ref, ep_ref, w2_ref, out_ref,
                  m_ref, l_ref, acc_ref, h_ref, weff_ref):
    i = pl.program_id(0)
    first = sinfo_ref[0]
    last = sinfo_ref[1]

    @pl.when(i == 0)
    def _init():
        m_ref[...] = jnp.full_like(m_ref, NEG)
        l_ref[...] = jnp.zeros_like(l_ref)
        acc_ref[...] = jnp.zeros_like(acc_ref)
        q = jax.lax.dot_general(zs_ref[...], wv_ref[...],
                                (((1,), (1,)), ((), ())),
                                preferred_element_type=jnp.float32)
        weff_ref[...] = jax.lax.dot_general(
            q, wu_ref[...], (((1,), (0,)), ((), ())),
            preferred_element_type=jnp.float32) * (H ** -0.5)

    @pl.when((i < N_P1) & (first + i <= last))
    def _phase1():
        c = first + i
        s = jax.lax.dot_general(weff_ref[...], zn_ref[...],
                                (((1,), (1,)), ((), ())),
                                preferred_element_type=jnp.float32)
        row = c * CHUNK + jax.lax.broadcasted_iota(jnp.int32, (B, CHUNK), 1)
        mask = (row >= st_ref[...]) & (row < en_ref[...])
        s = jnp.where(mask, s, NEG)

        m_prev = m_ref[...]
        m_new = jnp.maximum(m_prev, jnp.max(s, axis=1, keepdims=True))
        p = jnp.exp(s - m_new)
        corr = jnp.exp(m_prev - m_new)
        l_ref[...] = l_ref[...] * corr + jnp.sum(p, axis=1, keepdims=True)
        acc_ref[...] = acc_ref[...] * corr + jax.lax.dot_general(
            p, ep_ref[...], (((1,), (0,)), ((), ())),
            preferred_element_type=jnp.float32)
        m_ref[...] = m_new

    @pl.when(i == N_P1 - 1)
    def _finalize():
        nonempty = en_ref[...] > st_ref[...]
        evx = jnp.where(nonempty, acc_ref[...] / l_ref[...], 0.0)
        h = jax.lax.dot_general(evx, w1_ref[...], (((1,), (1,)), ((), ())),
                                preferred_element_type=jnp.float32) + b1_ref[...]
        h_ref[...] = 0.5 * h * (1.0 + jax.lax.erf(h * (2.0 ** -0.5)))

    @pl.when(i >= N_P1)
    def _phase2():
        out_ref[...] = jax.lax.dot_general(
            h_ref[...], w2_ref[...], (((1,), (1,)), ((), ())),
            preferred_element_type=jnp.float32) + b2_ref[...]


def kernel(Z_self, Z_neigh_flat, E_pair_flat, ptr, Wv, Wu, W1, b1, W2, b2):
    st = ptr[:B].reshape(B, 1)
    en = ptr[1:].reshape(B, 1)
    first = ptr[0] // CHUNK
    last = jnp.maximum(jnp.maximum(ptr[B] - 1, 0) // CHUNK, first)
    sinfo = jnp.stack([first, last]).astype(jnp.int32)

    def _p1_map(i, sinfo):
        return (jnp.minimum(sinfo[0] + i, sinfo[1]), 0)

    def _p2_map(i, sinfo):
        return (jnp.maximum(i - N_P1, 0), 0)

    out = pl.pallas_call(
        _fused_kernel,
        grid_spec=pltpu.PrefetchScalarGridSpec(
            num_scalar_prefetch=1,
            grid=(N_P1 + N_P2,),
            in_specs=[
                pl.BlockSpec((B, 1), lambda i, sinfo: (0, 0)),
                pl.BlockSpec((B, 1), lambda i, sinfo: (0, 0)),
                pl.BlockSpec((B, D_Z), lambda i, sinfo: (0, 0)),
                pl.BlockSpec((H, D_Z), lambda i, sinfo: (0, 0)),
                pl.BlockSpec((H, D_Z), lambda i, sinfo: (0, 0)),
                pl.BlockSpec((4 * D_LM, D_PAIR), lambda i, sinfo: (0, 0)),
                pl.BlockSpec((1, 4 * D_LM), lambda i, sinfo: (0, 0)),
                pl.BlockSpec((1, BLK), lambda i, sinfo: (0, jnp.maximum(i - N_P1, 0))),
                pl.BlockSpec((CHUNK, D_Z), _p1_map),
                pl.BlockSpec((CHUNK, D_PAIR), _p1_map),
                pl.BlockSpec((BLK, 4 * D_LM), _p2_map),
            ],
            out_specs=pl.BlockSpec((B, BLK), lambda i, sinfo: (0, jnp.maximum(i - N_P1, 0))),
            scratch_shapes=[
                pltpu.VMEM((B, 1), jnp.float32),
                pltpu.VMEM((B, 1), jnp.float32),
                pltpu.VMEM((B, D_PAIR), jnp.float32),
                pltpu.VMEM((B, 4 * D_LM), jnp.float32),
                pltpu.VMEM((B, D_Z), jnp.float32),
            ],
        ),
        out_shape=jax.ShapeDtypeStruct((B, M * D_LM), jnp.float32),
    )(sinfo, st, en, Z_self, Wv, Wu, W1, b1.reshape(1, -1), b2.reshape(1, -1),
      Z_neigh_flat, E_pair_flat, W2)

    return out.reshape(B, M, D_LM)


# speedup vs baseline: 1.0563x; 1.0563x over previous
import jax
import jax.numpy as jnp
from jax.experimental import pallas as pl
from jax.experimental.pallas import tpu as pltpu

B, TOTAL, D_Z, D_PAIR, D_LM, M, H = 16, 32768, 128, 128, 768, 16, 128

CHUNK = 2048
N_P1 = TOTAL // CHUNK
NEG = -1e30

BLK = 768
N_P2 = (M * D_LM) // BLK


def _fused_kernel(st_ref, en_ref, zs_ref, wv_ref, wu_ref, w1_ref, b1_ref,
                  b2_ref, zn_ref, ep_ref, w2_ref, out_ref,
                  m_ref, l_ref, acc_ref, h_ref, weff_ref):
    i = pl.program_id(0)

    @pl.when(i == 0)
    def _init():
        m_ref[...] = jnp.full_like(m_ref, NEG)
        l_ref[...] = jnp.zeros_like(l_ref)
        acc_ref[...] = jnp.zeros_like(acc_ref)
        q = jax.lax.dot_general(zs_ref[...], wv_ref[...],
                                (((1,), (1,)), ((), ())),
                                preferred_element_type=jnp.float32)
        weff_ref[...] = jax.lax.dot_general(
            q, wu_ref[...], (((1,), (0,)), ((), ())),
            preferred_element_type=jnp.float32) * (H ** -0.5)

    @pl.when(i < N_P1)
    def _phase1():
        s = jax.lax.dot_general(weff_ref[...], zn_ref[...], (((1,), (1,)), ((), ())),
                                preferred_element_type=jnp.float32)
        row = i * CHUNK + jax.lax.broadcasted_iota(jnp.int32, (B, CHUNK), 1)
        mask = (row >= st_ref[...]) & (row < en_ref[...])
        s = jnp.where(mask, s, NEG)

        m_prev = m_ref[...]
        m_new = jnp.maximum(m_prev, jnp.max(s, axis=1, keepdims=True))
        p = jnp.exp(s - m_new)
        corr = jnp.exp(m_prev - m_new)
        l_ref[...] = l_ref[...] * corr + jnp.sum(p, axis=1, keepdims=True)
        acc_ref[...] = acc_ref[...] * corr + jax.lax.dot_general(
            p, ep_ref[...], (((1,), (0,)), ((), ())),
            preferred_element_type=jnp.float32)
        m_ref[...] = m_new

        @pl.when(i == N_P1 - 1)
        def _finalize():
            nonempty = en_ref[...] > st_ref[...]
            evx = jnp.where(nonempty, acc_ref[...] / l_ref[...], 0.0)
            h = jax.lax.dot_general(evx, w1_ref[...], (((1,), (1,)), ((), ())),
                                    preferred_element_type=jnp.float32) + b1_ref[...]
            h_ref[...] = 0.5 * h * (1.0 + jax.lax.erf(h * (2.0 ** -0.5)))

    @pl.when(i >= N_P1)
    def _phase2():
        out_ref[...] = jax.lax.dot_general(
            h_ref[...], w2_ref[...], (((1,), (1,)), ((), ())),
            preferred_element_type=jnp.float32) + b2_ref[...]


def kernel(Z_self, Z_neigh_flat, E_pair_flat, ptr, Wv, Wu, W1, b1, W2, b2):
    st = ptr[:B].reshape(B, 1)
    en = ptr[1:].reshape(B, 1)

    out = pl.pallas_call(
        _fused_kernel,
        grid=(N_P1 + N_P2,),
        in_specs=[
            pl.BlockSpec((B, 1), lambda i: (0, 0)),
            pl.BlockSpec((B, 1), lambda i: (0, 0)),
            pl.BlockSpec((B, D_Z), lambda i: (0, 0)),
            pl.BlockSpec((H, D_Z), lambda i: (0, 0)),
            pl.BlockSpec((H, D_Z), lambda i: (0, 0)),
            pl.BlockSpec((4 * D_LM, D_PAIR), lambda i: (0, 0)),
            pl.BlockSpec((1, 4 * D_LM), lambda i: (0, 0)),
            pl.BlockSpec((1, BLK), lambda i: (0, jnp.maximum(i - N_P1, 0))),
            pl.BlockSpec((CHUNK, D_Z), lambda i: (jnp.minimum(i, N_P1 - 1), 0)),
            pl.BlockSpec((CHUNK, D_PAIR), lambda i: (jnp.minimum(i, N_P1 - 1), 0)),
            pl.BlockSpec((BLK, 4 * D_LM), lambda i: (jnp.maximum(i - N_P1, 0), 0)),
        ],
        out_specs=pl.BlockSpec((B, BLK), lambda i: (0, jnp.maximum(i - N_P1, 0))),
        out_shape=jax.ShapeDtypeStruct((B, M * D_LM), jnp.float32),
        scratch_shapes=[
            pltpu.VMEM((B, 1), jnp.float32),
            pltpu.VMEM((B, 1), jnp.float32),
            pltpu.VMEM((B, D_PAIR), jnp.float32),
            pltpu.VMEM((B, 4 * D_LM), jnp.float32),
            pltpu.VMEM((B, D_Z), jnp.float32),
        ],
    )(st, en, Z_self, Wv, Wu, W1, b1.reshape(1, -1), b2.reshape(1, -1),
      Z_neigh_flat, E_pair_flat, W2)

    return out.reshape(B, M, D_LM)


# speedup vs baseline: 1.1355x; 1.0749x over previous
import jax
import jax.numpy as jnp
from jax.experimental import pallas as pl
from jax.experimental.pallas import tpu as pltpu

B, TOTAL, D_Z, D_PAIR, D_LM, M, H = 16, 32768, 128, 128, 768, 16, 128

CHUNK = 4096
N_P1 = TOTAL // CHUNK
NEG = -1e30

BLK = 768
N_P2 = (M * D_LM) // BLK


def _fused_kernel(st_ref, en_ref, zs_ref, wv_ref, wu_ref, w1_ref, b1_ref,
                  b2_ref, zn_ref, ep_ref, w2_ref, out_ref,
                  m_ref, l_ref, acc_ref, h_ref, weff_ref):
    i = pl.program_id(0)

    @pl.when(i == 0)
    def _init():
        m_ref[...] = jnp.full_like(m_ref, NEG)
        l_ref[...] = jnp.zeros_like(l_ref)
        acc_ref[...] = jnp.zeros_like(acc_ref)
        q = jax.lax.dot_general(zs_ref[...], wv_ref[...],
                                (((1,), (1,)), ((), ())),
                                preferred_element_type=jnp.float32)
        weff_ref[...] = jax.lax.dot_general(
            q, wu_ref[...], (((1,), (0,)), ((), ())),
            preferred_element_type=jnp.float32) * (H ** -0.5)

    @pl.when(i < N_P1)
    def _phase1():
        s = jax.lax.dot_general(weff_ref[...], zn_ref[...], (((1,), (1,)), ((), ())),
                                preferred_element_type=jnp.float32)
        row = i * CHUNK + jax.lax.broadcasted_iota(jnp.int32, (B, CHUNK), 1)
        mask = (row >= st_ref[...]) & (row < en_ref[...])
        s = jnp.where(mask, s, NEG)

        m_prev = m_ref[...]
        m_new = jnp.maximum(m_prev, jnp.max(s, axis=1, keepdims=True))
        p = jnp.exp(s - m_new)
        corr = jnp.exp(m_prev - m_new)
        l_ref[...] = l_ref[...] * corr + jnp.sum(p, axis=1, keepdims=True)
        acc_ref[...] = acc_ref[...] * corr + jax.lax.dot_general(
            p, ep_ref[...], (((1,), (0,)), ((), ())),
            preferred_element_type=jnp.float32)
        m_ref[...] = m_new

        @pl.when(i == N_P1 - 1)
        def _finalize():
            nonempty = en_ref[...] > st_ref[...]
            evx = jnp.where(nonempty, acc_ref[...] / l_ref[...], 0.0)
            h = jax.lax.dot_general(evx, w1_ref[...], (((1,), (1,)), ((), ())),
                                    preferred_element_type=jnp.float32) + b1_ref[...]
            h_ref[...] = 0.5 * h * (1.0 + jax.lax.erf(h * (2.0 ** -0.5)))

    @pl.when(i >= N_P1)
    def _phase2():
        out_ref[...] = jax.lax.dot_general(
            h_ref[...], w2_ref[...], (((1,), (1,)), ((), ())),
            preferred_element_type=jnp.float32) + b2_ref[...]


def kernel(Z_self, Z_neigh_flat, E_pair_flat, ptr, Wv, Wu, W1, b1, W2, b2):
    st = ptr[:B].reshape(B, 1)
    en = ptr[1:].reshape(B, 1)

    out = pl.pallas_call(
        _fused_kernel,
        grid=(N_P1 + N_P2,),
        in_specs=[
            pl.BlockSpec((B, 1), lambda i: (0, 0)),
            pl.BlockSpec((B, 1), lambda i: (0, 0)),
            pl.BlockSpec((B, D_Z), lambda i: (0, 0)),
            pl.BlockSpec((H, D_Z), lambda i: (0, 0)),
            pl.BlockSpec((H, D_Z), lambda i: (0, 0)),
            pl.BlockSpec((4 * D_LM, D_PAIR), lambda i: (0, 0)),
            pl.BlockSpec((1, 4 * D_LM), lambda i: (0, 0)),
            pl.BlockSpec((1, BLK), lambda i: (0, jnp.maximum(i - N_P1, 0))),
            pl.BlockSpec((CHUNK, D_Z), lambda i: (jnp.minimum(i, N_P1 - 1), 0)),
            pl.BlockSpec((CHUNK, D_PAIR), lambda i: (jnp.minimum(i, N_P1 - 1), 0)),
            pl.BlockSpec((BLK, 4 * D_LM), lambda i: (jnp.maximum(i - N_P1, 0), 0)),
        ],
        out_specs=pl.BlockSpec((B, BLK), lambda i: (0, jnp.maximum(i - N_P1, 0))),
        out_shape=jax.ShapeDtypeStruct((B, M * D_LM), jnp.float32),
        scratch_shapes=[
            pltpu.VMEM((B, 1), jnp.float32),
            pltpu.VMEM((B, 1), jnp.float32),
            pltpu.VMEM((B, D_PAIR), jnp.float32),
            pltpu.VMEM((B, 4 * D_LM), jnp.float32),
            pltpu.VMEM((B, D_Z), jnp.float32),
        ],
    )(st, en, Z_self, Wv, Wu, W1, b1.reshape(1, -1), b2.reshape(1, -1),
      Z_neigh_flat, E_pair_flat, W2)

    return out.reshape(B, M, D_LM)


# speedup vs baseline: 1.1521x; 1.0147x over previous
import jax
import jax.numpy as jnp
from jax.experimental import pallas as pl
from jax.experimental.pallas import tpu as pltpu

B, TOTAL, D_Z, D_PAIR, D_LM, M, H = 16, 32768, 128, 128, 768, 16, 128

CHUNK = 8192
N_P1 = TOTAL // CHUNK
NEG = -1e30

BLK = 768
N_P2 = (M * D_LM) // BLK


def _fused_kernel(st_ref, en_ref, zs_ref, wv_ref, wu_ref, w1_ref, b1_ref,
                  b2_ref, zn_ref, ep_ref, w2_ref, out_ref,
                  m_ref, l_ref, acc_ref, h_ref, weff_ref):
    i = pl.program_id(0)

    @pl.when(i == 0)
    def _init():
        m_ref[...] = jnp.full_like(m_ref, NEG)
        l_ref[...] = jnp.zeros_like(l_ref)
        acc_ref[...] = jnp.zeros_like(acc_ref)
        q = jax.lax.dot_general(zs_ref[...], wv_ref[...],
                                (((1,), (1,)), ((), ())),
                                preferred_element_type=jnp.float32)
        weff_ref[...] = jax.lax.dot_general(
            q, wu_ref[...], (((1,), (0,)), ((), ())),
            preferred_element_type=jnp.float32) * (H ** -0.5)

    @pl.when(i < N_P1)
    def _phase1():
        s = jax.lax.dot_general(weff_ref[...], zn_ref[...], (((1,), (1,)), ((), ())),
                                preferred_element_type=jnp.float32)
        row = i * CHUNK + jax.lax.broadcasted_iota(jnp.int32, (B, CHUNK), 1)
        mask = (row >= st_ref[...]) & (row < en_ref[...])
        s = jnp.where(mask, s, NEG)

        m_prev = m_ref[...]
        m_new = jnp.maximum(m_prev, jnp.max(s, axis=1, keepdims=True))
        p = jnp.exp(s - m_new)
        corr = jnp.exp(m_prev - m_new)
        l_ref[...] = l_ref[...] * corr + jnp.sum(p, axis=1, keepdims=True)
        acc_ref[...] = acc_ref[...] * corr + jax.lax.dot_general(
            p, ep_ref[...], (((1,), (0,)), ((), ())),
            preferred_element_type=jnp.float32)
        m_ref[...] = m_new

        @pl.when(i == N_P1 - 1)
        def _finalize():
            nonempty = en_ref[...] > st_ref[...]
            evx = jnp.where(nonempty, acc_ref[...] / l_ref[...], 0.0)
            h = jax.lax.dot_general(evx, w1_ref[...], (((1,), (1,)), ((), ())),
                                    preferred_element_type=jnp.float32) + b1_ref[...]
            h_ref[...] = 0.5 * h * (1.0 + jax.lax.erf(h * (2.0 ** -0.5)))

    @pl.when(i >= N_P1)
    def _phase2():
        out_ref[...] = jax.lax.dot_general(
            h_ref[...], w2_ref[...], (((1,), (1,)), ((), ())),
            preferred_element_type=jnp.float32) + b2_ref[...]


def kernel(Z_self, Z_neigh_flat, E_pair_flat, ptr, Wv, Wu, W1, b1, W2, b2):
    st = ptr[:B].reshape(B, 1)
    en = ptr[1:].reshape(B, 1)

    out = pl.pallas_call(
        _fused_kernel,
        grid=(N_P1 + N_P2,),
        in_specs=[
            pl.BlockSpec((B, 1), lambda i: (0, 0)),
            pl.BlockSpec((B, 1), lambda i: (0, 0)),
            pl.BlockSpec((B, D_Z), lambda i: (0, 0)),
            pl.BlockSpec((H, D_Z), lambda i: (0, 0)),
            pl.BlockSpec((H, D_Z), lambda i: (0, 0)),
            pl.BlockSpec((4 * D_LM, D_PAIR), lambda i: (0, 0)),
            pl.BlockSpec((1, 4 * D_LM), lambda i: (0, 0)),
            pl.BlockSpec((1, BLK), lambda i: (0, jnp.maximum(i - N_P1, 0))),
            pl.BlockSpec((CHUNK, D_Z), lambda i: (jnp.minimum(i, N_P1 - 1), 0)),
            pl.BlockSpec((CHUNK, D_PAIR), lambda i: (jnp.minimum(i, N_P1 - 1), 0)),
            pl.BlockSpec((BLK, 4 * D_LM), lambda i: (jnp.maximum(i - N_P1, 0), 0)),
        ],
        out_specs=pl.BlockSpec((B, BLK), lambda i: (0, jnp.maximum(i - N_P1, 0))),
        out_shape=jax.ShapeDtypeStruct((B, M * D_LM), jnp.float32),
        scratch_shapes=[
            pltpu.VMEM((B, 1), jnp.float32),
            pltpu.VMEM((B, 1), jnp.float32),
            pltpu.VMEM((B, D_PAIR), jnp.float32),
            pltpu.VMEM((B, 4 * D_LM), jnp.float32),
            pltpu.VMEM((B, D_Z), jnp.float32),
        ],
    )(st, en, Z_self, Wv, Wu, W1, b1.reshape(1, -1), b2.reshape(1, -1),
      Z_neigh_flat, E_pair_flat, W2)

    return out.reshape(B, M, D_LM)
